# baseline (speedup 1.0000x reference)
import jax
import jax.numpy as jnp
from jax.experimental import pallas as pl
from jax.experimental.pallas import tpu as pltpu

_ROWS = 1024


def _mix_body(tok_ref, pos_ref, m_ref, me_ref, tw_ref, pw_ref, mw_ref, out_ref):
    tw = tw_ref[0, 0]
    pw = pw_ref[0, 0]
    mw = mw_ref[0, 0]
    masked_row = mw * me_ref[0, :]
    m = m_ref[0, 0, :][:, None]
    out_ref[...] = tw * tok_ref[...] + pw * pos_ref[...] + m * masked_row[None, :]


def kernel(token_embeds, mask_embeds, position_embeds, mask_inds,
           token_weight, mask_weight, position_weight):
    B, S, D = token_embeds.shape
    N = B * S
    R = _ROWS
    nblk = N // R

    tok2 = token_embeds.reshape(N, D)
    pos2 = position_embeds.reshape(N, D)
    maskf = mask_inds.reshape(nblk, 1, R).astype(jnp.float32)
    me2 = mask_embeds.reshape(1, D)
    tw2 = token_weight.reshape(1, 1)
    pw2 = position_weight.reshape(1, 1)
    mw2 = mask_weight.reshape(1, 1)

    out = pl.pallas_call(
        _mix_body,
        grid=(nblk,),
        in_specs=[
            pl.BlockSpec((R, D), lambda i: (i, 0)),
            pl.BlockSpec((R, D), lambda i: (i, 0)),
            pl.BlockSpec((1, 1, R), lambda i: (i, 0, 0)),
            pl.BlockSpec((1, D), lambda i: (0, 0)),
            pl.BlockSpec((1, 1), lambda i: (0, 0)),
            pl.BlockSpec((1, 1), lambda i: (0, 0)),
            pl.BlockSpec((1, 1), lambda i: (0, 0)),
        ],
        out_specs=pl.BlockSpec((R, D), lambda i: (i, 0)),
        out_shape=jax.ShapeDtypeStruct((N, D), jnp.float32),
        compiler_params=pltpu.CompilerParams(
            dimension_semantics=("arbitrary",),
        ),
    )(tok2, pos2, maskf, me2, tw2, pw2, mw2)
    return out.reshape(B, S, D)


# speedup vs baseline: 1.0074x; 1.0074x over previous
import jax
import jax.numpy as jnp
from jax.experimental import pallas as pl
from jax.experimental.pallas import tpu as pltpu

_ROWS = 512


def _mix_body(tok_ref, pos_ref, m_ref, me_ref, tw_ref, pw_ref, mw_ref, out_ref):
    tw = tw_ref[0, 0]
    pw = pw_ref[0, 0]
    mw = mw_ref[0, 0]
    masked_row = mw * me_ref[0, :]
    m = m_ref[0, 0, :][:, None]
    out_ref[...] = tw * tok_ref[...] + pw * pos_ref[...] + m * masked_row[None, :]


def kernel(token_embeds, mask_embeds, position_embeds, mask_inds,
           token_weight, mask_weight, position_weight):
    B, S, D = token_embeds.shape
    N = B * S
    R = _ROWS
    nblk = N // R

    tok2 = token_embeds.reshape(N, D)
    pos2 = position_embeds.reshape(N, D)
    maskf = mask_inds.reshape(nblk, 1, R).astype(jnp.float32)
    me2 = mask_embeds.reshape(1, D)
    tw2 = token_weight.reshape(1, 1)
    pw2 = position_weight.reshape(1, 1)
    mw2 = mask_weight.reshape(1, 1)

    out = pl.pallas_call(
        _mix_body,
        grid=(nblk,),
        in_specs=[
            pl.BlockSpec((R, D), lambda i: (i, 0)),
            pl.BlockSpec((R, D), lambda i: (i, 0)),
            pl.BlockSpec((1, 1, R), lambda i: (i, 0, 0)),
            pl.BlockSpec((1, D), lambda i: (0, 0)),
            pl.BlockSpec((1, 1), lambda i: (0, 0)),
            pl.BlockSpec((1, 1), lambda i: (0, 0)),
            pl.BlockSpec((1, 1), lambda i: (0, 0)),
        ],
        out_specs=pl.BlockSpec((R, D), lambda i: (i, 0)),
        out_shape=jax.ShapeDtypeStruct((N, D), jnp.float32),
        compiler_params=pltpu.CompilerParams(
            dimension_semantics=("parallel",),
        ),
    )(tok2, pos2, maskf, me2, tw2, pw2, mw2)
    return out.reshape(B, S, D)


# speedup vs baseline: 1.0212x; 1.0137x over previous
import jax
import jax.numpy as jnp
from jax.experimental import pallas as pl
from jax.experimental.pallas import tpu as pltpu

_ROWS = 256
_NBUF = 4


def _make_body(N, D, R, NBUF):
    nblk = N // R

    def body(tok_hbm, pos_hbm, m_ref, me_ref, tw_ref, pw_ref, mw_ref,
             out_hbm, tok_buf, pos_buf, out_buf, sems):
        tw = tw_ref[0]
        pw = pw_ref[0]
        mw = mw_ref[0]
        mrow = mw * me_ref[0, :]

        def in_copies(i, slot):
            return (
                pltpu.make_async_copy(
                    tok_hbm.at[pl.ds(i * R, R), :], tok_buf.at[slot],
                    sems.at[slot, 0]),
                pltpu.make_async_copy(
                    pos_hbm.at[pl.ds(i * R, R), :], pos_buf.at[slot],
                    sems.at[slot, 1]),
            )

        def out_copy(i, slot):
            return pltpu.make_async_copy(
                out_buf.at[slot], out_hbm.at[pl.ds(i * R, R), :],
                sems.at[slot, 2])

        for k in range(min(NBUF - 1, nblk)):
            for c in in_copies(k, k % NBUF):
                c.start()

        def step(i, carry):
            slot = jax.lax.rem(i, NBUF)
            ctok, cpos = in_copies(i, slot)
            ctok.wait()
            cpos.wait()

            @pl.when(i >= NBUF)
            def _():
                out_copy(i - NBUF, slot).wait()

            m = m_ref[i, 0, :][:, None]
            out_buf[slot] = (tw * tok_buf[slot] + pw * pos_buf[slot]
                             + m * mrow[None, :])
            out_copy(i, slot).start()

            nxt = i + NBUF - 1
            @pl.when(nxt < nblk)
            def _():
                for c in in_copies(nxt, jax.lax.rem(nxt, NBUF)):
                    c.start()

            return carry

        jax.lax.fori_loop(0, nblk, step, 0)

        tail = min(NBUF, nblk)
        for k in range(tail):
            j = nblk - tail + k
            out_copy(j, j % NBUF).wait()

    return body


def kernel(token_embeds, mask_embeds, position_embeds, mask_inds,
           token_weight, mask_weight, position_weight):
    B, S, D = token_embeds.shape
    N = B * S
    R = _ROWS
    nblk = N // R

    tok2 = token_embeds.reshape(N, D)
    pos2 = position_embeds.reshape(N, D)
    maskf = mask_inds.reshape(nblk, 1, R).astype(jnp.float32)
    me2 = mask_embeds.reshape(1, D)

    out = pl.pallas_call(
        _make_body(N, D, R, _NBUF),
        in_specs=[
            pl.BlockSpec(memory_space=pltpu.HBM),
            pl.BlockSpec(memory_space=pltpu.HBM),
            pl.BlockSpec(memory_space=pltpu.VMEM),
            pl.BlockSpec(memory_space=pltpu.VMEM),
            pl.BlockSpec(memory_space=pltpu.SMEM),
            pl.BlockSpec(memory_space=pltpu.SMEM),
            pl.BlockSpec(memory_space=pltpu.SMEM),
        ],
        out_specs=pl.BlockSpec(memory_space=pltpu.HBM),
        out_shape=jax.ShapeDtypeStruct((N, D), jnp.float32),
        scratch_shapes=[
            pltpu.VMEM((_NBUF, R, D), jnp.float32),
            pltpu.VMEM((_NBUF, R, D), jnp.float32),
            pltpu.VMEM((_NBUF, R, D), jnp.float32),
            pltpu.SemaphoreType.DMA((_NBUF, 3)),
        ],
    )(tok2, pos2, maskf, me2, token_weight, position_weight, mask_weight)
    return out.reshape(B, S, D)


# speedup vs baseline: 1.0214x; 1.0001x over previous
import jax
import jax.numpy as jnp
from jax.experimental import pallas as pl
from jax.experimental.pallas import tpu as pltpu

_ROWS = 128
_NBUF = 8


def _make_body(N, D, R, NBUF):
    nblk = N // R

    def body(tok_hbm, pos_hbm, m_ref, me_ref, tw_ref, pw_ref, mw_ref,
             out_hbm, tok_buf, pos_buf, out_buf, sems):
        tw = tw_ref[0]
        pw = pw_ref[0]
        mw = mw_ref[0]
        mrow = mw * me_ref[0, :]

        def in_copies(i, slot):
            return (
                pltpu.make_async_copy(
                    tok_hbm.at[pl.ds(i * R, R), :], tok_buf.at[slot],
                    sems.at[slot, 0]),
                pltpu.make_async_copy(
                    pos_hbm.at[pl.ds(i * R, R), :], pos_buf.at[slot],
                    sems.at[slot, 1]),
            )

        def out_copy(i, slot):
            return pltpu.make_async_copy(
                out_buf.at[slot], out_hbm.at[pl.ds(i * R, R), :],
                sems.at[slot, 2])

        for k in range(min(NBUF - 1, nblk)):
            for c in in_copies(k, k % NBUF):
                c.start()

        def step(i, carry):
            slot = jax.lax.rem(i, NBUF)
            ctok, cpos = in_copies(i, slot)
            ctok.wait()
            cpos.wait()

            @pl.when(i >= NBUF)
            def _():
                out_copy(i - NBUF, slot).wait()

            m = m_ref[i, 0, :][:, None]
            out_buf[slot] = (tw * tok_buf[slot] + pw * pos_buf[slot]
                             + m * mrow[None, :])
            out_copy(i, slot).start()

            nxt = i + NBUF - 1
            @pl.when(nxt < nblk)
            def _():
                for c in in_copies(nxt, jax.lax.rem(nxt, NBUF)):
                    c.start()

            return carry

        jax.lax.fori_loop(0, nblk, step, 0)

        tail = min(NBUF, nblk)
        for k in range(tail):
            j = nblk - tail + k
            out_copy(j, j % NBUF).wait()

    return body


def kernel(token_embeds, mask_embeds, position_embeds, mask_inds,
           token_weight, mask_weight, position_weight):
    B, S, D = token_embeds.shape
    N = B * S
    R = _ROWS
    nblk = N // R

    tok2 = token_embeds.reshape(N, D)
    pos2 = position_embeds.reshape(N, D)
    maskf = mask_inds.reshape(nblk, 1, R).astype(jnp.float32)
    me2 = mask_embeds.reshape(1, D)

    out = pl.pallas_call(
        _make_body(N, D, R, _NBUF),
        in_specs=[
            pl.BlockSpec(memory_space=pltpu.HBM),
            pl.BlockSpec(memory_space=pltpu.HBM),
            pl.BlockSpec(memory_space=pltpu.VMEM),
            pl.BlockSpec(memory_space=pltpu.VMEM),
            pl.BlockSpec(memory_space=pltpu.SMEM),
            pl.BlockSpec(memory_space=pltpu.SMEM),
            pl.BlockSpec(memory_space=pltpu.SMEM),
        ],
        out_specs=pl.BlockSpec(memory_space=pltpu.HBM),
        out_shape=jax.ShapeDtypeStruct((N, D), jnp.float32),
        scratch_shapes=[
            pltpu.VMEM((_NBUF, R, D), jnp.float32),
            pltpu.VMEM((_NBUF, R, D), jnp.float32),
            pltpu.VMEM((_NBUF, R, D), jnp.float32),
            pltpu.SemaphoreType.DMA((_NBUF, 3)),
        ],
    )(tok2, pos2, maskf, me2, token_weight, position_weight, mask_weight)
    return out.reshape(B, S, D)
